# baseline (speedup 1.0000x reference)
import jax
import jax.numpy as jnp
from jax.experimental import pallas as pl
from jax.experimental.pallas import tpu as pltpu

NUM_CLASSES = 100000
BLOCK_R = 64


def _onehot_block(users_ref, out_ref):
    cols = jax.lax.broadcasted_iota(jnp.int32, out_ref.shape, 1)
    out_ref[:, :] = (users_ref[:, :] == cols).astype(jnp.float32)


def kernel(users):
    b = users.shape[0]
    users2 = users.reshape(b, 1)
    return pl.pallas_call(
        _onehot_block,
        grid=(b // BLOCK_R,),
        in_specs=[pl.BlockSpec((BLOCK_R, 1), lambda i: (i, 0))],
        out_specs=pl.BlockSpec((BLOCK_R, NUM_CLASSES), lambda i: (i, 0)),
        out_shape=jax.ShapeDtypeStruct((b, NUM_CLASSES), jnp.float32),
    )(users2)


# speedup vs baseline: 1.0008x; 1.0008x over previous
import jax
import jax.numpy as jnp
from jax.experimental import pallas as pl
from jax.experimental.pallas import tpu as pltpu

NUM_CLASSES = 100000
BLOCK_R = 16
NBUF = 8


def _onehot_body(nsteps, users_ref, out_hbm, scratch, sems):
    j = pl.program_id(0)
    slot = jax.lax.rem(j, NBUF)

    @pl.when(j >= NBUF)
    def _wait_prev():
        prev = j - NBUF
        pltpu.make_async_copy(
            scratch.at[slot],
            out_hbm.at[pl.ds(prev * BLOCK_R, BLOCK_R), :],
            sems.at[slot],
        ).wait()

    cols = jax.lax.broadcasted_iota(jnp.int32, (BLOCK_R, NUM_CLASSES), 1)
    u = users_ref[pl.ds(j * BLOCK_R, BLOCK_R), :]
    scratch[slot] = (u == cols).astype(jnp.float32)

    pltpu.make_async_copy(
        scratch.at[slot],
        out_hbm.at[pl.ds(j * BLOCK_R, BLOCK_R), :],
        sems.at[slot],
    ).start()

    @pl.when(j == nsteps - 1)
    def _drain():
        for k in range(min(NBUF, nsteps)):
            step = nsteps - 1 - k
            pltpu.make_async_copy(
                scratch.at[step % NBUF],
                out_hbm.at[pl.ds(step * BLOCK_R, BLOCK_R), :],
                sems.at[step % NBUF],
            ).wait()


def kernel(users):
    b = users.shape[0]
    nsteps = b // BLOCK_R
    users2 = users.reshape(b, 1)
    import functools

    return pl.pallas_call(
        functools.partial(_onehot_body, nsteps),
        grid=(nsteps,),
        in_specs=[pl.BlockSpec(memory_space=pltpu.MemorySpace.VMEM)],
        out_specs=pl.BlockSpec(memory_space=pltpu.MemorySpace.HBM),
        out_shape=jax.ShapeDtypeStruct((b, NUM_CLASSES), jnp.float32),
        scratch_shapes=[
            pltpu.VMEM((NBUF, BLOCK_R, NUM_CLASSES), jnp.float32),
            pltpu.SemaphoreType.DMA((NBUF,)),
        ],
        compiler_params=pltpu.CompilerParams(
            vmem_limit_bytes=110 * 1024 * 1024,
        ),
    )(users2)


# speedup vs baseline: 1.0032x; 1.0024x over previous
import functools

import jax
import jax.numpy as jnp
from jax.experimental import pallas as pl
from jax.experimental.pallas import tpu as pltpu

NUM_CLASSES = 100000
BLOCK_R = 16
NBUF = 8


def _onehot_body(nsteps, users_ref, out_hbm, *bufs_and_sems):
    bufs = bufs_and_sems[:NBUF]
    sems = bufs_and_sems[NBUF:]
    j = pl.program_id(0)
    slot = jax.lax.rem(j, NBUF)

    cols = jax.lax.broadcasted_iota(jnp.int32, (BLOCK_R, NUM_CLASSES), 1)
    u = users_ref[pl.ds(j * BLOCK_R, BLOCK_R), :]
    val = (u == cols).astype(jnp.float32)

    def _wait_prev(k):
        pltpu.make_async_copy(
            bufs[k],
            out_hbm.at[pl.ds((j - NBUF) * BLOCK_R, BLOCK_R), :],
            sems[k],
        ).wait()

    def _fill_and_send(k):
        bufs[k][...] = val
        pltpu.make_async_copy(
            bufs[k],
            out_hbm.at[pl.ds(j * BLOCK_R, BLOCK_R), :],
            sems[k],
        ).start()

    for k in range(NBUF):
        pl.when(jnp.logical_and(slot == k, j >= NBUF))(
            functools.partial(_wait_prev, k)
        )
        pl.when(slot == k)(functools.partial(_fill_and_send, k))

    @pl.when(j == nsteps - 1)
    def _drain():
        for step in range(max(0, nsteps - NBUF), nsteps):
            pltpu.make_async_copy(
                bufs[step % NBUF],
                out_hbm.at[pl.ds(step * BLOCK_R, BLOCK_R), :],
                sems[step % NBUF],
            ).wait()


def kernel(users):
    b = users.shape[0]
    nsteps = b // BLOCK_R
    users2 = users.reshape(b, 1)
    scratch = [pltpu.VMEM((BLOCK_R, NUM_CLASSES), jnp.float32)] * NBUF
    dsems = [pltpu.SemaphoreType.DMA] * NBUF
    return pl.pallas_call(
        functools.partial(_onehot_body, nsteps),
        grid=(nsteps,),
        in_specs=[pl.BlockSpec(memory_space=pltpu.MemorySpace.VMEM)],
        out_specs=pl.BlockSpec(memory_space=pltpu.MemorySpace.HBM),
        out_shape=jax.ShapeDtypeStruct((b, NUM_CLASSES), jnp.float32),
        scratch_shapes=scratch + dsems,
        compiler_params=pltpu.CompilerParams(
            vmem_limit_bytes=110 * 1024 * 1024,
        ),
    )(users2)
